# baseline (speedup 1.0000x reference)
import functools

import jax
import jax.numpy as jnp
from jax import lax
from jax.experimental import pallas as pl
from jax.experimental.pallas import tpu as pltpu
from jax.experimental.pallas import tpu_sc as plsc

DIM = 32
B = 4096
LC = 200
LQ = 20

NC = 2
NS = 16
NW = NC * NS

SUB = 128
NBT = B // SUB
NCU = LC // 8 * NBT
NQU = LQ * NBT
CU_W = NCU // NW
QU_W = NQU // NW
NBC = 3 * NBT


def _body(wcV, pcV, kcV, wqV, pqV, kqV,
          W_word, W_pos, W_kg,
          ctxV, qV,
          idx_v, rows_v, stg, gsem, wsem):
  wid = lax.axis_index("s") * NC + lax.axis_index("c")
  iota = lax.iota(jnp.int32, 16)

  for t, (table, IV) in enumerate(((W_word, wcV), (W_pos, pcV), (W_kg, kcV))):

    @pl.loop(0, CU_W)
    def _(u, table=table, IV=IV, t=t):
      ug = wid * CU_W + u
      lt = ug // NBT
      bt = ug % NBT
      pltpu.sync_copy(IV.at[ug], idx_v)
      copies = []
      for j in range(8):
        copies.append(
            pltpu.async_copy(table.at[idx_v.at[j]],
                             rows_v.at[pl.ds(j * SUB, SUB)], gsem))
      for c in copies:
        c.wait()

      @pl.loop(0, 32)
      def _(k):
        li = k // 4
        do = k % 4

        @pl.loop(0, 8)
        def _(di, li=li, do=do, k=k):
          for g in range(8):
            rv = li * SUB + g * 16 + iota
            cv = jnp.full((16,), do * 8 + di, jnp.int32)
            stg[k, pl.ds(di * SUB + g * 16, 16)] = (
                plsc.load_gather(rows_v, [rv, cv]))

      @pl.loop(0, 32)
      def _(k, lt=lt, bt=bt, t=t):
        pltpu.async_copy(stg.at[k],
                         ctxV.at[(NBT * lt + k) * NBC + NBT * t + bt], wsem)

      @pl.loop(0, 32)
      def _(k, lt=lt, bt=bt, t=t):
        pltpu.make_async_copy(
            stg.at[k],
            ctxV.at[(NBT * lt + k) * NBC + NBT * t + bt], wsem).wait()

  for t, (table, QV) in enumerate(((W_word, wqV), (W_pos, pqV), (W_kg, kqV))):

    @pl.loop(0, QU_W)
    def _(u, table=table, QV=QV, t=t):
      ug = wid * QU_W + u
      l = ug // NBT
      bt = ug % NBT
      pltpu.sync_copy(QV.at[ug], idx_v.at[0])
      pltpu.async_copy(table.at[idx_v.at[0]],
                       rows_v.at[pl.ds(0, SUB)], gsem).wait()

      @pl.loop(0, 4)
      def _(k):
        @pl.loop(0, 8)
        def _(di, k=k):
          for g in range(8):
            rv = g * 16 + iota
            cv = jnp.full((16,), k * 8 + di, jnp.int32)
            stg[k, pl.ds(di * SUB + g * 16, 16)] = (
                plsc.load_gather(rows_v, [rv, cv]))

      @pl.loop(0, 4)
      def _(k, l=l, bt=bt, t=t):
        pltpu.async_copy(stg.at[k],
                         qV.at[(4 * l + k) * NBC + NBT * t + bt], wsem)

      @pl.loop(0, 4)
      def _(k, l=l, bt=bt, t=t):
        pltpu.make_async_copy(
            stg.at[k],
            qV.at[(4 * l + k) * NBC + NBT * t + bt], wsem).wait()


@jax.jit
def _run(wcV, pcV, kcV, wqV, pqV, kqV, W_word, W_pos, W_kg):
  mesh = plsc.VectorSubcoreMesh(core_axis_name="c", subcore_axis_name="s")
  ctxV, qV = pl.kernel(
      _body,
      out_type=(
          jax.ShapeDtypeStruct((LC * 4 * NBC, 8 * SUB), jnp.float32),
          jax.ShapeDtypeStruct((LQ * 4 * NBC, 8 * SUB), jnp.float32),
      ),
      mesh=mesh,
      compiler_params=pltpu.CompilerParams(use_tc_tiling_on_sc=False,
                                           needs_layout_passes=False),
      scratch_types=[
          pltpu.VMEM((8, SUB), jnp.int32),
          pltpu.VMEM((8 * SUB, DIM), jnp.float32),
          pltpu.VMEM((32, 8 * SUB), jnp.float32),
          pltpu.SemaphoreType.DMA,
          pltpu.SemaphoreType.DMA,
      ],
  )(wcV, pcV, kcV, wqV, pqV, kqV, W_word, W_pos, W_kg)
  return ctxV, qV


def _ctx_idx_view(x):
  return (x.T.reshape(LC // 8, 8, NBT, SUB)
          .transpose(0, 2, 1, 3).reshape(NCU, 8, SUB))


def _q_idx_view(x):
  return x.T.reshape(NQU, SUB)


def _out_view(y, L):
  return (y.reshape(L, 4, NBC, 8, SUB).transpose(0, 1, 3, 2, 4)
          .reshape(L, DIM, 3 * B).transpose(2, 0, 1))


def kernel(word_context, word_question, kg_context, kg_question,
           pos_context, pos_question, W_word, W_pos, W_kg):
  ctxV, qV = _run(
      _ctx_idx_view(word_context),
      _ctx_idx_view(pos_context),
      _ctx_idx_view(kg_context),
      _q_idx_view(word_question),
      _q_idx_view(pos_question),
      _q_idx_view(kg_question),
      W_word, W_pos, W_kg)
  return (_out_view(ctxV, LC), _out_view(qV, LQ))


# speedup vs baseline: 1.0893x; 1.0893x over previous
import functools

import jax
import jax.numpy as jnp
from jax import lax
from jax.experimental import pallas as pl
from jax.experimental.pallas import tpu as pltpu
from jax.experimental.pallas import tpu_sc as plsc

DIM = 32
B = 4096
LC = 200
LQ = 20

NC = 2
NS = 16
NW = NC * NS

SUB = 128
NBT = B // SUB
NCU = LC // 8 * NBT
NQU = LQ * NBT
CU_W = NCU // NW
QU_W = NQU // NW
NBC = 3 * NBT


def _body(wcV, pcV, kcV, wqV, pqV, kqV,
          W_word, W_pos, W_kg,
          ctxV, qV,
          idx0, idx1, rows0, rows1, stg, sem0, sem1, wsem):
  wid = lax.axis_index("s") * NC + lax.axis_index("c")
  iota = lax.iota(jnp.int32, 16)
  rvg = [g * 16 + iota for g in range(8)]
  idx = (idx0, idx1)
  rows = (rows0, rows1)
  sem = (sem0, sem1)

  def make_job(table, IV, out, t, n_units, n_sub, nk):
    def unit(u):
      return wid * n_units + u

    def fire(u, p):
      ug = unit(u)
      if n_sub == 8:
        pltpu.sync_copy(IV.at[ug], idx[p])
      else:
        pltpu.sync_copy(IV.at[ug], idx[p].at[0])
      for j in range(n_sub):
        pltpu.async_copy(table.at[idx[p].at[j]],
                         rows[p].at[pl.ds(j * SUB, SUB)], sem[p])

    def process(u, p):
      ug = unit(u)
      bt = ug % NBT
      ro = ug // NBT
      for j in range(n_sub):
        pltpu.make_async_copy(table.at[idx[p].at[j]],
                              rows[p].at[pl.ds(j * SUB, SUB)], sem[p]).wait()

      @pl.loop(0, nk)
      def _(k, p=p):
        base = (k // 4) * SUB
        cv0 = (k % 4) * 8

        @pl.loop(0, 8)
        def _(di, base=base, cv0=cv0, k=k):
          cv = jnp.full((16,), cv0 + di, jnp.int32)
          for g in range(8):
            stg[k, pl.ds(di * SUB + g * 16, 16)] = plsc.load_gather(
                rows[p], [base + rvg[g], cv])

      @pl.loop(0, nk)
      def _(k, ro=ro, bt=bt):
        pltpu.async_copy(stg.at[k],
                         out.at[(4 * ro * (nk // 4) + k) * NBC + NBT * t + bt],
                         wsem)

      @pl.loop(0, nk)
      def _(k, ro=ro, bt=bt):
        pltpu.make_async_copy(
            stg.at[k],
            out.at[(4 * ro * (nk // 4) + k) * NBC + NBT * t + bt],
            wsem).wait()

    fire(0, 0)

    @pl.loop(0, n_units // 2)
    def _(tt):
      fire(2 * tt + 1, 1)
      process(2 * tt, 0)

      @pl.when(2 * tt + 2 < n_units)
      def _():
        fire(2 * tt + 2, 0)

      process(2 * tt + 1, 1)

    if n_units % 2 == 1:
      process(n_units - 1, 0)

  for t, (table, IV) in enumerate(((W_word, wcV), (W_pos, pcV), (W_kg, kcV))):
    make_job(table, IV, ctxV, t, CU_W, 8, 32)
  for t, (table, QV) in enumerate(((W_word, wqV), (W_pos, pqV), (W_kg, kqV))):
    make_job(table, QV, qV, t, QU_W, 1, 4)


@jax.jit
def _run(wcV, pcV, kcV, wqV, pqV, kqV, W_word, W_pos, W_kg):
  mesh = plsc.VectorSubcoreMesh(core_axis_name="c", subcore_axis_name="s")
  ctxV, qV = pl.kernel(
      _body,
      out_type=(
          jax.ShapeDtypeStruct((LC * 4 * NBC, 8 * SUB), jnp.float32),
          jax.ShapeDtypeStruct((LQ * 4 * NBC, 8 * SUB), jnp.float32),
      ),
      mesh=mesh,
      compiler_params=pltpu.CompilerParams(use_tc_tiling_on_sc=False,
                                           needs_layout_passes=False),
      scratch_types=[
          pltpu.VMEM((8, SUB), jnp.int32),
          pltpu.VMEM((8, SUB), jnp.int32),
          pltpu.VMEM((8 * SUB, DIM), jnp.float32),
          pltpu.VMEM((8 * SUB, DIM), jnp.float32),
          pltpu.VMEM((32, 8 * SUB), jnp.float32),
          pltpu.SemaphoreType.DMA,
          pltpu.SemaphoreType.DMA,
          pltpu.SemaphoreType.DMA,
      ],
  )(wcV, pcV, kcV, wqV, pqV, kqV, W_word, W_pos, W_kg)
  return ctxV, qV


def _ctx_idx_view(x):
  return (x.T.reshape(LC // 8, 8, NBT, SUB)
          .transpose(0, 2, 1, 3).reshape(NCU, 8, SUB))


def _q_idx_view(x):
  return x.T.reshape(NQU, SUB)


def _out_view(y, L):
  return (y.reshape(L, 4, NBC, 8, SUB).transpose(0, 1, 3, 2, 4)
          .reshape(L, DIM, 3 * B).transpose(2, 0, 1))


def kernel(word_context, word_question, kg_context, kg_question,
           pos_context, pos_question, W_word, W_pos, W_kg):
  ctxV, qV = _run(
      _ctx_idx_view(word_context),
      _ctx_idx_view(pos_context),
      _ctx_idx_view(kg_context),
      _q_idx_view(word_question),
      _q_idx_view(pos_question),
      _q_idx_view(kg_question),
      W_word, W_pos, W_kg)
  return (_out_view(ctxV, LC), _out_view(qV, LQ))


# speedup vs baseline: 1.1078x; 1.0171x over previous
import functools

import jax
import jax.numpy as jnp
from jax import lax
from jax.experimental import pallas as pl
from jax.experimental.pallas import tpu as pltpu
from jax.experimental.pallas import tpu_sc as plsc

DIM = 32
B = 4096
LC = 200
LQ = 20

NC = 2
NS = 16
NW = NC * NS

SUB = 128
NBT = B // SUB
NCU = LC // 8 * NBT
NQU = LQ * NBT
CU_W = NCU // NW
QU_W = NQU // NW
NBC = 3 * NBT


def _body(wcV, pcV, kcV, wqV, pqV, kqV,
          W_word, W_pos, W_kg,
          ctxV, qV,
          idx0, idx1, rows0, rows1, stg0, stg1, sem0, sem1, wsem):
  wid = lax.axis_index("s") * NC + lax.axis_index("c")
  iota = lax.iota(jnp.int32, 16)
  rvg = [g * 16 + iota for g in range(8)]
  idx = (idx0, idx1)
  rows = (rows0, rows1)
  stg = (stg0, stg1)
  sem = (sem0, sem1)

  def ctx_job(table, IV, t):
    def fh(u):
      return wid * CU_W + u // 2, u % 2

    def fire(u, p):
      f, h = fh(u)
      pltpu.sync_copy(IV.at[f, pl.ds(4 * h, 4)], idx[p])
      for j in range(4):
        pltpu.async_copy(table.at[idx[p].at[j]],
                         rows[p].at[pl.ds(j * SUB, SUB)], sem[p])

    def tile_row(u, k):
      f, h = fh(u)
      return (NBT * (f // NBT) + 16 * h + k) * NBC + NBT * t + (f % NBT)

    def proc(u, p):
      for j in range(4):
        pltpu.make_async_copy(table.at[idx[p].at[j]],
                              rows[p].at[pl.ds(j * SUB, SUB)], sem[p]).wait()

      @pl.loop(0, 16)
      def _(k, p=p):
        base = (k // 4) * SUB
        cv0 = (k % 4) * 8

        @pl.loop(0, 8)
        def _(di, base=base, cv0=cv0, k=k, p=p):
          cv = jnp.full((16,), cv0 + di, jnp.int32)
          for g in range(8):
            stg[p][k, pl.ds(di * SUB + g * 16, 16)] = plsc.load_gather(
                rows[p], [base + rvg[g], cv])

      @pl.loop(0, 16)
      def _(k, u=u, p=p):
        pltpu.async_copy(stg[p].at[k], ctxV.at[tile_row(u, k)], wsem)

    def drain(u, p):
      @pl.loop(0, 16)
      def _(k, u=u, p=p):
        pltpu.make_async_copy(stg[p].at[k], ctxV.at[tile_row(u, k)],
                              wsem).wait()

    n = 2 * CU_W
    fire(0, 0)

    @pl.loop(0, n // 2)
    def _(tt):
      fire(2 * tt + 1, 1)
      proc(2 * tt, 0)

      @pl.when(2 * tt + 2 < n)
      def _():
        fire(2 * tt + 2, 0)

      @pl.when(tt > 0)
      def _():
        drain(2 * tt - 1, 1)

      proc(2 * tt + 1, 1)
      drain(2 * tt, 0)

    drain(n - 1, 1)

  def q_job(table, QV, t):
    def fire(u, p):
      ug = wid * QU_W + u
      pltpu.sync_copy(QV.at[ug], idx[p].at[0])
      pltpu.async_copy(table.at[idx[p].at[0]],
                       rows[p].at[pl.ds(0, SUB)], sem[p])

    def tile_row(u, k):
      ug = wid * QU_W + u
      return (4 * (ug // NBT) + k) * NBC + NBT * t + ug % NBT

    def proc(u, p):
      pltpu.make_async_copy(table.at[idx[p].at[0]],
                            rows[p].at[pl.ds(0, SUB)], sem[p]).wait()

      @pl.loop(0, 4)
      def _(k, p=p):
        @pl.loop(0, 8)
        def _(di, k=k, p=p):
          cv = jnp.full((16,), (k % 4) * 8 + di, jnp.int32)
          for g in range(8):
            stg[p][k, pl.ds(di * SUB + g * 16, 16)] = plsc.load_gather(
                rows[p], [rvg[g], cv])

      @pl.loop(0, 4)
      def _(k, u=u, p=p):
        pltpu.async_copy(stg[p].at[k], qV.at[tile_row(u, k)], wsem)

    def drain(u, p):
      @pl.loop(0, 4)
      def _(k, u=u, p=p):
        pltpu.make_async_copy(stg[p].at[k], qV.at[tile_row(u, k)],
                              wsem).wait()

    n = QU_W
    fire(0, 0)

    @pl.loop(0, n // 2)
    def _(tt):
      fire(2 * tt + 1, 1)
      proc(2 * tt, 0)

      @pl.when(2 * tt + 2 < n)
      def _():
        fire(2 * tt + 2, 0)

      @pl.when(tt > 0)
      def _():
        drain(2 * tt - 1, 1)

      proc(2 * tt + 1, 1)
      drain(2 * tt, 0)

    drain(n - 1, 1)

  for t, (table, IV) in enumerate(((W_word, wcV), (W_pos, pcV), (W_kg, kcV))):
    ctx_job(table, IV, t)
  for t, (table, QV) in enumerate(((W_word, wqV), (W_pos, pqV), (W_kg, kqV))):
    q_job(table, QV, t)


@jax.jit
def _run(wcV, pcV, kcV, wqV, pqV, kqV, W_word, W_pos, W_kg):
  mesh = plsc.VectorSubcoreMesh(core_axis_name="c", subcore_axis_name="s")
  ctxV, qV = pl.kernel(
      _body,
      out_type=(
          jax.ShapeDtypeStruct((LC * 4 * NBC, 8 * SUB), jnp.float32),
          jax.ShapeDtypeStruct((LQ * 4 * NBC, 8 * SUB), jnp.float32),
      ),
      mesh=mesh,
      compiler_params=pltpu.CompilerParams(use_tc_tiling_on_sc=False,
                                           needs_layout_passes=False),
      scratch_types=[
          pltpu.VMEM((4, SUB), jnp.int32),
          pltpu.VMEM((4, SUB), jnp.int32),
          pltpu.VMEM((4 * SUB, DIM), jnp.float32),
          pltpu.VMEM((4 * SUB, DIM), jnp.float32),
          pltpu.VMEM((16, 8 * SUB), jnp.float32),
          pltpu.VMEM((16, 8 * SUB), jnp.float32),
          pltpu.SemaphoreType.DMA,
          pltpu.SemaphoreType.DMA,
          pltpu.SemaphoreType.DMA,
      ],
  )(wcV, pcV, kcV, wqV, pqV, kqV, W_word, W_pos, W_kg)
  return ctxV, qV


def _ctx_idx_view(x):
  return (x.T.reshape(LC // 8, 8, NBT, SUB)
          .transpose(0, 2, 1, 3).reshape(NCU, 8, SUB))


def _q_idx_view(x):
  return x.T.reshape(NQU, SUB)


def _out_view(y, L):
  return (y.reshape(L, 4, NBC, 8, SUB).transpose(0, 1, 3, 2, 4)
          .reshape(L, DIM, 3 * B).transpose(2, 0, 1))


def kernel(word_context, word_question, kg_context, kg_question,
           pos_context, pos_question, W_word, W_pos, W_kg):
  ctxV, qV = _run(
      _ctx_idx_view(word_context),
      _ctx_idx_view(pos_context),
      _ctx_idx_view(kg_context),
      _q_idx_view(word_question),
      _q_idx_view(pos_question),
      _q_idx_view(kg_question),
      W_word, W_pos, W_kg)
  return (_out_view(ctxV, LC), _out_view(qV, LQ))
